# baseline (speedup 1.0000x reference)
import jax
import jax.numpy as jnp
from jax import lax
from jax.experimental import pallas as pl
from jax.experimental.pallas import tpu as pltpu

_PARENTS = (-1, 0, 1, 2, 0, 4, 5, 0, 7, 8, 9, 8, 11, 12, 8, 14, 15)
_N = 17
_H = 8
_Z = 256
_W2 = tuple(1 if c % 2 == 1 else -1 for c in range(_N))

_BB = 128


def _body(fa_ref, attn_ref, pts_ref, vec_ref, out_ref):
    bb = attn_ref.shape[3]
    scores = jnp.sum(attn_ref[...], axis=(1, 2))
    node_i = lax.broadcasted_iota(jnp.int32, (_N, bb), 0)

    m1 = jnp.max(scores, axis=0, keepdims=True)
    idx0 = jnp.min(jnp.where(scores == m1, node_i, _N), axis=0, keepdims=True)
    masked = jnp.where(node_i == idx0, -jnp.inf, scores)
    m2 = jnp.max(masked, axis=0, keepdims=True)
    idx1 = jnp.min(jnp.where(masked == m2, node_i, _N), axis=0, keepdims=True)

    idx0_t = jnp.swapaxes(idx0, 0, 1)
    idx1_t = jnp.swapaxes(idx1, 0, 1)

    def cls_w(idx):
        return (
            (idx == 0).astype(jnp.float32),
            ((idx >= 1) & (idx <= 3)).astype(jnp.float32),
            ((idx >= 4) & (idx <= 6)).astype(jnp.float32),
            (idx >= 7).astype(jnp.float32),
        )

    a = cls_w(idx0_t)
    b = cls_w(idx1_t)
    w = [a[k] + b[k] for k in range(4)]

    def pcol(c):
        return pts_ref[c]

    def vcol(c):
        return vec_ref[:, c, :]

    u0 = pcol(0)
    u1 = pcol(1) + vcol(0)
    u4 = pcol(4) - vcol(3)
    u7 = pcol(7) + vcol(6)
    upd = 0.5 * (w[0] * u0 + w[1] * u1 + w[2] * u4 + w[3] * u7)

    fa = fa_ref[0]
    nv = [None] * _N
    nv[0] = upd
    out_ref[0] = pcol(0) + fa * upd
    for c in range(1, _N):
        v = nv[_PARENTS[c]] - float(_W2[c]) * vcol(c - 1)
        nv[c] = v
        out_ref[c] = pcol(c) + fa * v


@jax.jit
def kernel(points, vectors, attntion_scors, Fa):
    bsz = points.shape[0]
    attn_t = attntion_scors.transpose(2, 3, 1, 0)
    pts_t = points.transpose(1, 0, 2)
    grid = (bsz // _BB,)
    out_t = pl.pallas_call(
        _body,
        grid=grid,
        in_specs=[
            pl.BlockSpec(memory_space=pltpu.SMEM),
            pl.BlockSpec((_N, _N, _H, _BB), lambda i: (0, 0, 0, i)),
            pl.BlockSpec((_N, _BB, _Z), lambda i: (0, i, 0)),
            pl.BlockSpec((_BB, 16, _Z), lambda i: (i, 0, 0)),
        ],
        out_specs=pl.BlockSpec((_N, _BB, _Z), lambda i: (0, i, 0)),
        out_shape=jax.ShapeDtypeStruct((_N, bsz, _Z), jnp.float32),
    )(Fa, attn_t, pts_t, vectors)
    return out_t.transpose(1, 0, 2)


# speedup vs baseline: 1.0438x; 1.0438x over previous
import jax
import jax.numpy as jnp
from jax import lax
from jax.experimental import pallas as pl
from jax.experimental.pallas import tpu as pltpu

_PARENTS = (-1, 0, 1, 2, 0, 4, 5, 0, 7, 8, 9, 8, 11, 12, 8, 14, 15)
_N = 17
_H = 8
_Z = 256
_W2 = tuple(1 if c % 2 == 1 else -1 for c in range(_N))

_BB = 256


def _body(fa_ref, attn_ref, pts_ref, vec_ref, out_ref):
    bb = attn_ref.shape[3]
    scores = jnp.sum(attn_ref[...], axis=(1, 2))
    node_i = lax.broadcasted_iota(jnp.int32, (_N, bb), 0)

    m1 = jnp.max(scores, axis=0, keepdims=True)
    idx0 = jnp.min(jnp.where(scores == m1, node_i, _N), axis=0, keepdims=True)
    masked = jnp.where(node_i == idx0, -jnp.inf, scores)
    m2 = jnp.max(masked, axis=0, keepdims=True)
    idx1 = jnp.min(jnp.where(masked == m2, node_i, _N), axis=0, keepdims=True)

    idx0_t = jnp.swapaxes(idx0, 0, 1)
    idx1_t = jnp.swapaxes(idx1, 0, 1)

    def cls_w(idx):
        return (
            (idx == 0).astype(jnp.float32),
            ((idx >= 1) & (idx <= 3)).astype(jnp.float32),
            ((idx >= 4) & (idx <= 6)).astype(jnp.float32),
            (idx >= 7).astype(jnp.float32),
        )

    a = cls_w(idx0_t)
    b = cls_w(idx1_t)
    w = [a[k] + b[k] for k in range(4)]

    def pcol(c):
        return pts_ref[c]

    def vcol(c):
        return vec_ref[:, c, :]

    u0 = pcol(0)
    u1 = pcol(1) + vcol(0)
    u4 = pcol(4) - vcol(3)
    u7 = pcol(7) + vcol(6)
    upd = 0.5 * (w[0] * u0 + w[1] * u1 + w[2] * u4 + w[3] * u7)

    fa = fa_ref[0]
    nv = [None] * _N
    nv[0] = upd
    out_ref[0] = pcol(0) + fa * upd
    for c in range(1, _N):
        v = nv[_PARENTS[c]] - float(_W2[c]) * vcol(c - 1)
        nv[c] = v
        out_ref[c] = pcol(c) + fa * v


@jax.jit
def kernel(points, vectors, attntion_scors, Fa):
    bsz = points.shape[0]
    attn_t = attntion_scors.transpose(2, 3, 1, 0)
    pts_t = points.transpose(1, 0, 2)
    grid = (bsz // _BB,)
    out_t = pl.pallas_call(
        _body,
        grid=grid,
        in_specs=[
            pl.BlockSpec(memory_space=pltpu.SMEM),
            pl.BlockSpec((_N, _N, _H, _BB), lambda i: (0, 0, 0, i)),
            pl.BlockSpec((_N, _BB, _Z), lambda i: (0, i, 0)),
            pl.BlockSpec((_BB, 16, _Z), lambda i: (i, 0, 0)),
        ],
        out_specs=pl.BlockSpec((_N, _BB, _Z), lambda i: (0, i, 0)),
        out_shape=jax.ShapeDtypeStruct((_N, bsz, _Z), jnp.float32),
    )(Fa, attn_t, pts_t, vectors)
    return out_t.transpose(1, 0, 2)
